# Initial kernel scaffold; baseline (speedup 1.0000x reference)
import jax
import jax.numpy as jnp
from jax.experimental import pallas as pl


def kernel(x, edge_index, edge_attr, u, batch, W1, b1, ln_g, ln_b, W2, b2):
    raise NotImplementedError("write your pallas kernel here")


# speedup vs baseline: 7.1118x; 7.1118x over previous
import functools

import jax
import jax.numpy as jnp
from jax import lax
from jax.experimental import pallas as pl
from jax.experimental.pallas import tpu as pltpu
from jax.experimental.pallas import tpu_sc as plsc

N_NODES = 10000
NODE_DIM = 256
NUM_GRAPHS = 128
G_DIM = 201
TAIL_DIM = 33
HIDDEN = 50
LANES = 16
GPT = 4
R = 64
FC = 64
STATS_W = 4 * NODE_DIM


def _sc_stats_body(x_hbm, starts_hbm, out_hbm, sbuf, xbuf, accv):
    c = lax.axis_index("c")
    s = lax.axis_index("s")
    wid = s * 2 + c
    g0 = wid * GPT
    pltpu.sync_copy(starts_hbm.at[pl.ds(wid * 8, LANES)], sbuf)
    sv = sbuf[...]
    bounds = [sv[i] for i in range(GPT + 1)]

    zeros = jnp.zeros((LANES,), jnp.float32)
    ninf = jnp.full((LANES,), -jnp.inf, jnp.float32)
    pinf = jnp.full((LANES,), jnp.inf, jnp.float32)

    for k in range(GPT):
        a = bounds[k]
        b = bounds[k + 1]
        n = b - a

        for q in range(NODE_DIM // LANES):
            accv[pl.ds(0 * NODE_DIM + q * LANES, LANES)] = zeros
            accv[pl.ds(1 * NODE_DIM + q * LANES, LANES)] = zeros
            accv[pl.ds(2 * NODE_DIM + q * LANES, LANES)] = ninf
            accv[pl.ds(3 * NODE_DIM + q * LANES, LANES)] = pinf

        nchunks = (n + R - 1) // R

        def chunk_body(j, carry, a=a, b=b):
            cs = a + j * R
            safe = jnp.minimum(cs, N_NODES - R)
            delta = cs - safe
            pltpu.sync_copy(x_hbm.at[pl.ds(safe * NODE_DIM, R * NODE_DIM)], xbuf)
            lim = jnp.minimum(R, b - cs)
            for fc in range(NODE_DIM // FC):
                col0 = fc * FC
                acc = []
                for st in range(4):
                    for q in range(4):
                        acc.append(accv[pl.ds(st * NODE_DIM + col0 + q * LANES, LANES)])

                def row_body(i, acc, delta=delta, col0=col0):
                    acc = list(acc)
                    roff = (delta + i) * NODE_DIM + col0
                    for q in range(4):
                        v = xbuf[pl.ds(roff + q * LANES, LANES)]
                        acc[0 + q] = acc[0 + q] + v
                        acc[4 + q] = acc[4 + q] + v * v
                        acc[8 + q] = jnp.maximum(acc[8 + q], v)
                        acc[12 + q] = jnp.minimum(acc[12 + q], v)
                    return tuple(acc)

                acc = lax.fori_loop(0, lim, row_body, tuple(acc))
                for st in range(4):
                    for q in range(4):
                        accv[pl.ds(st * NODE_DIM + col0 + q * LANES, LANES)] = acc[st * 4 + q]
            return carry

        lax.fori_loop(0, nchunks, chunk_body, 0)

        nv = jnp.full((LANES,), n, jnp.int32)
        cc = jnp.maximum(nv.astype(jnp.float32), 1.0)
        inv = 1.0 / cc
        for q in range(NODE_DIM // LANES):
            sl_s = pl.ds(0 * NODE_DIM + q * LANES, LANES)
            sl_2 = pl.ds(1 * NODE_DIM + q * LANES, LANES)
            sm = accv[sl_s]
            s2 = accv[sl_2]
            mean = sm * inv
            var = s2 * inv - mean * mean
            accv[sl_s] = mean
            accv[sl_2] = var

        @pl.when(n == 0)
        def _():
            for q in range(NODE_DIM // LANES):
                accv[pl.ds(2 * NODE_DIM + q * LANES, LANES)] = zeros
                accv[pl.ds(3 * NODE_DIM + q * LANES, LANES)] = zeros
        pltpu.sync_copy(accv, out_hbm.at[pl.ds((g0 + k) * STATS_W, STATS_W)])


_sc_stats = pl.kernel(
    _sc_stats_body,
    out_type=jax.ShapeDtypeStruct((NUM_GRAPHS * STATS_W,), jnp.float32),
    mesh=plsc.VectorSubcoreMesh(core_axis_name="c", subcore_axis_name="s"),
    scratch_types=[
        pltpu.VMEM((LANES,), jnp.int32),
        pltpu.VMEM((R * NODE_DIM,), jnp.float32),
        pltpu.VMEM((STATS_W,), jnp.float32),
    ],
)


def _mlp_body(stats_ref, u_ref, W1_ref, b1_ref, g_ref, bb_ref, W2_ref, b2_ref,
              out_ref):
    stats = stats_ref[...]
    var = stats[:, NODE_DIM:2 * NODE_DIM]
    std = jnp.sqrt(jnp.maximum(var, 0.0) + 1e-5)
    aggr = jnp.concatenate(
        [stats[:, :NODE_DIM], std, stats[:, 2 * NODE_DIM:]], axis=1)
    h = jnp.dot(aggr, W1_ref[...], preferred_element_type=jnp.float32,
                precision=lax.Precision.HIGHEST) + b1_ref[...]
    alpha = 1.6732632423543772
    scale = 1.0507009873554805
    h = scale * jnp.where(h > 0, h, alpha * (jnp.exp(h) - 1.0))
    mu = jnp.mean(h, axis=-1, keepdims=True)
    v = jnp.mean((h - mu) ** 2, axis=-1, keepdims=True)
    h = (h - mu) / jnp.sqrt(v + 1e-5) * g_ref[...] + bb_ref[...]
    head = jnp.dot(h, W2_ref[...], preferred_element_type=jnp.float32,
                   precision=lax.Precision.HIGHEST) + b2_ref[...]
    out_ref[:, :G_DIM] = head
    out_ref[:, G_DIM:] = u_ref[:, G_DIM - TAIL_DIM:]


_mlp = pl.pallas_call(
    _mlp_body,
    out_shape=jax.ShapeDtypeStruct((NUM_GRAPHS, G_DIM + TAIL_DIM), jnp.float32),
)


def kernel(x, edge_index, edge_attr, u, batch, W1, b1, ln_g, ln_b, W2, b2):
    gids = jnp.arange(NUM_GRAPHS + 1, dtype=batch.dtype)
    starts = jnp.searchsorted(batch, gids, side="left").astype(jnp.int32)
    widx = 4 * jnp.arange(32, dtype=jnp.int32)[:, None] + \
        jnp.arange(8, dtype=jnp.int32)[None, :]
    starts_win = starts[jnp.clip(widx, 0, NUM_GRAPHS)].reshape(-1)
    starts_win = jnp.concatenate(
        [starts_win, jnp.full((8,), N_NODES, jnp.int32)])
    stats = _sc_stats(jnp.reshape(x, (-1,)), starts_win)
    stats = stats.reshape(NUM_GRAPHS, STATS_W)
    return _mlp(stats, u, W1, b1, ln_g, ln_b, W2, b2)


# speedup vs baseline: 7.3662x; 1.0358x over previous
import functools

import jax
import jax.numpy as jnp
from jax import lax
from jax.experimental import pallas as pl
from jax.experimental.pallas import tpu as pltpu
from jax.experimental.pallas import tpu_sc as plsc

N_NODES = 10000
NODE_DIM = 256
NUM_GRAPHS = 128
G_DIM = 201
TAIL_DIM = 33
HIDDEN = 50
LANES = 16
GPT = 4
R = 64
FC = 64
NFC = NODE_DIM // FC
STATS_W = 4 * NODE_DIM


def _sc_stats_body(x_hbm, starts_hbm, out_hbm, sbuf, xbuf0, xbuf1, accv,
                   sem0, sem1):
    c = lax.axis_index("c")
    s = lax.axis_index("s")
    wid = s * 2 + c
    g0 = wid * GPT
    pltpu.sync_copy(starts_hbm.at[pl.ds(wid * 8, LANES)], sbuf)
    sv = sbuf[...]
    bounds = [sv[i] for i in range(GPT + 1)]
    A = bounds[0]
    B = bounds[GPT]

    zeros = jnp.zeros((LANES,), jnp.float32)
    ninf = jnp.full((LANES,), -jnp.inf, jnp.float32)
    pinf = jnp.full((LANES,), jnp.inf, jnp.float32)

    for k in range(GPT):
        for q in range(NODE_DIM // LANES):
            accv[pl.ds(k * STATS_W + 0 * NODE_DIM + q * LANES, LANES)] = zeros
            accv[pl.ds(k * STATS_W + 1 * NODE_DIM + q * LANES, LANES)] = zeros
            accv[pl.ds(k * STATS_W + 2 * NODE_DIM + q * LANES, LANES)] = ninf
            accv[pl.ds(k * STATS_W + 3 * NODE_DIM + q * LANES, LANES)] = pinf

    jlo = A // R
    nchunks = (B + R - 1) // R - jlo

    def start_dma(j, buf, sem):
        cs = (jlo + j) * R
        safe = jnp.minimum(cs, N_NODES - R)
        pltpu.make_async_copy(x_hbm.at[pl.ds(safe, R)], buf, sem).start()

    def process(j, buf, sem):
        cs = (jlo + j) * R
        safe = jnp.minimum(cs, N_NODES - R)
        delta = cs - safe
        pltpu.make_async_copy(x_hbm.at[pl.ds(safe, R)], buf, sem).wait()
        for k in range(GPT):
            lo = jnp.maximum(cs, bounds[k]) - cs
            hi = jnp.minimum(cs + R, bounds[k + 1]) - cs

            @pl.when(hi > lo)
            def _(lo=lo, hi=hi, k=k):
                for fc in range(NFC):
                    col0 = fc * FC
                    base = k * STATS_W
                    acc = []
                    for st in range(4):
                        for q in range(4):
                            acc.append(accv[pl.ds(
                                base + st * NODE_DIM + col0 + q * LANES,
                                LANES)])

                    def row_body(t, acc, col0=col0):
                        acc = list(acc)
                        r = delta + t
                        for q in range(4):
                            v = buf[r, pl.ds(col0 + q * LANES, LANES)]
                            acc[0 + q] = acc[0 + q] + v
                            acc[4 + q] = acc[4 + q] + v * v
                            acc[8 + q] = jnp.maximum(acc[8 + q], v)
                            acc[12 + q] = jnp.minimum(acc[12 + q], v)
                        return tuple(acc)

                    acc = lax.fori_loop(lo, hi, row_body, tuple(acc))
                    for st in range(4):
                        for q in range(4):
                            accv[pl.ds(
                                base + st * NODE_DIM + col0 + q * LANES,
                                LANES)] = acc[st * 4 + q]

    @pl.when(nchunks > 0)
    def _():
        start_dma(0, xbuf0, sem0)

    npairs = (nchunks + 1) // 2

    def pair_body(p, carry):
        j0 = 2 * p
        j1 = j0 + 1

        @pl.when(j1 < nchunks)
        def _():
            start_dma(j1, xbuf1, sem1)

        process(j0, xbuf0, sem0)

        @pl.when(j1 < nchunks)
        def _():
            @pl.when(j1 + 1 < nchunks)
            def _():
                start_dma(j1 + 1, xbuf0, sem0)

            process(j1, xbuf1, sem1)

        return carry

    lax.fori_loop(0, npairs, pair_body, 0)

    for k in range(GPT):
        n = bounds[k + 1] - bounds[k]
        nv = jnp.full((LANES,), n, jnp.int32)
        cc = jnp.maximum(nv.astype(jnp.float32), 1.0)
        inv = 1.0 / cc
        base = k * STATS_W
        for q in range(NODE_DIM // LANES):
            sl_s = pl.ds(base + 0 * NODE_DIM + q * LANES, LANES)
            sl_2 = pl.ds(base + 1 * NODE_DIM + q * LANES, LANES)
            sm = accv[sl_s]
            s2 = accv[sl_2]
            mean = sm * inv
            var = s2 * inv - mean * mean
            accv[sl_s] = mean
            accv[sl_2] = var

        @pl.when(n == 0)
        def _(base=base):
            for q in range(NODE_DIM // LANES):
                accv[pl.ds(base + 2 * NODE_DIM + q * LANES, LANES)] = zeros
                accv[pl.ds(base + 3 * NODE_DIM + q * LANES, LANES)] = zeros

        pltpu.sync_copy(accv.at[pl.ds(base, STATS_W)], out_hbm.at[g0 + k])


_sc_stats = pl.kernel(
    _sc_stats_body,
    out_type=jax.ShapeDtypeStruct((NUM_GRAPHS, STATS_W), jnp.float32),
    mesh=plsc.VectorSubcoreMesh(core_axis_name="c", subcore_axis_name="s"),
    scratch_types=[
        pltpu.VMEM((LANES,), jnp.int32),
        pltpu.VMEM((R, NODE_DIM), jnp.float32),
        pltpu.VMEM((R, NODE_DIM), jnp.float32),
        pltpu.VMEM((GPT * STATS_W,), jnp.float32),
        pltpu.SemaphoreType.DMA,
        pltpu.SemaphoreType.DMA,
    ],
)


def _mlp_body(stats_ref, u_ref, W1_ref, b1_ref, g_ref, bb_ref, W2_ref, b2_ref,
              out_ref):
    stats = stats_ref[...]
    var = stats[:, NODE_DIM:2 * NODE_DIM]
    std = jnp.sqrt(jnp.maximum(var, 0.0) + 1e-5)
    aggr = jnp.concatenate(
        [stats[:, :NODE_DIM], std, stats[:, 2 * NODE_DIM:]], axis=1)
    h = jnp.dot(aggr, W1_ref[...], preferred_element_type=jnp.float32,
                precision=lax.Precision.HIGHEST) + b1_ref[...]
    alpha = 1.6732632423543772
    scale = 1.0507009873554805
    h = scale * jnp.where(h > 0, h, alpha * (jnp.exp(h) - 1.0))
    mu = jnp.mean(h, axis=-1, keepdims=True)
    v = jnp.mean((h - mu) ** 2, axis=-1, keepdims=True)
    h = (h - mu) / jnp.sqrt(v + 1e-5) * g_ref[...] + bb_ref[...]
    head = jnp.dot(h, W2_ref[...], preferred_element_type=jnp.float32,
                   precision=lax.Precision.HIGHEST) + b2_ref[...]
    out_ref[:, :G_DIM] = head
    out_ref[:, G_DIM:] = u_ref[:, G_DIM - TAIL_DIM:]


_mlp = pl.pallas_call(
    _mlp_body,
    out_shape=jax.ShapeDtypeStruct((NUM_GRAPHS, G_DIM + TAIL_DIM), jnp.float32),
)


def kernel(x, edge_index, edge_attr, u, batch, W1, b1, ln_g, ln_b, W2, b2):
    gids = jnp.arange(NUM_GRAPHS + 1, dtype=batch.dtype)
    starts = jnp.searchsorted(batch, gids, side="left").astype(jnp.int32)
    widx = 4 * jnp.arange(32, dtype=jnp.int32)[:, None] + \
        jnp.arange(8, dtype=jnp.int32)[None, :]
    starts_win = starts[jnp.clip(widx, 0, NUM_GRAPHS)].reshape(-1)
    starts_win = jnp.concatenate(
        [starts_win, jnp.full((8,), N_NODES, jnp.int32)])
    stats = _sc_stats(x, starts_win)
    return _mlp(stats, u, W1, b1, ln_g, ln_b, W2, b2)


# speedup vs baseline: 12.9263x; 1.7548x over previous
import functools

import jax
import jax.numpy as jnp
from jax import lax
from jax.experimental import pallas as pl
from jax.experimental.pallas import tpu as pltpu
from jax.experimental.pallas import tpu_sc as plsc

N_NODES = 10000
NODE_DIM = 256
NUM_GRAPHS = 128
G_DIM = 201
TAIL_DIM = 33
HIDDEN = 50
LANES = 16
GPT = 4
R = 64
FC = 64
NFC = NODE_DIM // FC
STATS_W = 4 * NODE_DIM


def _sc_stats_body(x_hbm, starts_hbm, out_hbm, sbuf, xbuf0, xbuf1, accv,
                   sem0, sem1):
    c = lax.axis_index("c")
    s = lax.axis_index("s")
    wid = s * 2 + c
    g0 = wid * GPT
    pltpu.sync_copy(starts_hbm.at[pl.ds(wid * 8, LANES)], sbuf)
    sv = sbuf[...]
    bounds = [sv[i] for i in range(GPT + 1)]
    A = bounds[0]
    B = bounds[GPT]

    zeros = jnp.zeros((LANES,), jnp.float32)
    ninf = jnp.full((LANES,), -jnp.inf, jnp.float32)
    pinf = jnp.full((LANES,), jnp.inf, jnp.float32)

    for k in range(GPT):
        for q in range(NODE_DIM // LANES):
            accv[pl.ds(k * STATS_W + 0 * NODE_DIM + q * LANES, LANES)] = zeros
            accv[pl.ds(k * STATS_W + 1 * NODE_DIM + q * LANES, LANES)] = zeros
            accv[pl.ds(k * STATS_W + 2 * NODE_DIM + q * LANES, LANES)] = ninf
            accv[pl.ds(k * STATS_W + 3 * NODE_DIM + q * LANES, LANES)] = pinf

    jlo = A // R
    nchunks = (B + R - 1) // R - jlo

    def start_dma(j, buf, sem):
        cs = (jlo + j) * R
        safe = jnp.minimum(cs, N_NODES - R)
        pltpu.make_async_copy(x_hbm.at[pl.ds(safe, R)], buf, sem).start()

    def process(j, buf, sem):
        cs = (jlo + j) * R
        safe = jnp.minimum(cs, N_NODES - R)
        delta = cs - safe
        pltpu.make_async_copy(x_hbm.at[pl.ds(safe, R)], buf, sem).wait()
        for k in range(GPT):
            lo = jnp.maximum(cs, bounds[k]) - cs
            hi = jnp.minimum(cs + R, bounds[k + 1]) - cs

            @pl.when(hi > lo)
            def _(lo=lo, hi=hi, k=k):
                for fc in range(NFC):
                    col0 = fc * FC
                    base = k * STATS_W
                    acc = []
                    for st in range(4):
                        for q in range(4):
                            acc.append(accv[pl.ds(
                                base + st * NODE_DIM + col0 + q * LANES,
                                LANES)])

                    def row_body(t, acc, col0=col0):
                        acc = list(acc)
                        r = delta + t
                        for q in range(4):
                            v = buf[r, pl.ds(col0 + q * LANES, LANES)]
                            acc[0 + q] = acc[0 + q] + v
                            acc[4 + q] = acc[4 + q] + v * v
                            acc[8 + q] = jnp.maximum(acc[8 + q], v)
                            acc[12 + q] = jnp.minimum(acc[12 + q], v)
                        return tuple(acc)

                    acc = lax.fori_loop(lo, hi, row_body, tuple(acc))
                    for st in range(4):
                        for q in range(4):
                            accv[pl.ds(
                                base + st * NODE_DIM + col0 + q * LANES,
                                LANES)] = acc[st * 4 + q]

    @pl.when(nchunks > 0)
    def _():
        start_dma(0, xbuf0, sem0)

    npairs = (nchunks + 1) // 2

    def pair_body(p, carry):
        j0 = 2 * p
        j1 = j0 + 1

        @pl.when(j1 < nchunks)
        def _():
            start_dma(j1, xbuf1, sem1)

        process(j0, xbuf0, sem0)

        @pl.when(j1 < nchunks)
        def _():
            @pl.when(j1 + 1 < nchunks)
            def _():
                start_dma(j1 + 1, xbuf0, sem0)

            process(j1, xbuf1, sem1)

        return carry

    lax.fori_loop(0, npairs, pair_body, 0)

    for k in range(GPT):
        n = bounds[k + 1] - bounds[k]
        nv = jnp.full((LANES,), n, jnp.int32)
        cc = jnp.maximum(nv.astype(jnp.float32), 1.0)
        inv = 1.0 / cc
        base = k * STATS_W
        for q in range(NODE_DIM // LANES):
            sl_s = pl.ds(base + 0 * NODE_DIM + q * LANES, LANES)
            sl_2 = pl.ds(base + 1 * NODE_DIM + q * LANES, LANES)
            sm = accv[sl_s]
            s2 = accv[sl_2]
            mean = sm * inv
            var = s2 * inv - mean * mean
            accv[sl_s] = mean
            accv[sl_2] = var

        @pl.when(n == 0)
        def _(base=base):
            for q in range(NODE_DIM // LANES):
                accv[pl.ds(base + 2 * NODE_DIM + q * LANES, LANES)] = zeros
                accv[pl.ds(base + 3 * NODE_DIM + q * LANES, LANES)] = zeros

        pltpu.sync_copy(accv.at[pl.ds(base, STATS_W)], out_hbm.at[g0 + k])


_sc_stats = pl.kernel(
    _sc_stats_body,
    out_type=jax.ShapeDtypeStruct((NUM_GRAPHS, STATS_W), jnp.float32),
    mesh=plsc.VectorSubcoreMesh(core_axis_name="c", subcore_axis_name="s"),
    scratch_types=[
        pltpu.VMEM((LANES,), jnp.int32),
        pltpu.VMEM((R, NODE_DIM), jnp.float32),
        pltpu.VMEM((R, NODE_DIM), jnp.float32),
        pltpu.VMEM((GPT * STATS_W,), jnp.float32),
        pltpu.SemaphoreType.DMA,
        pltpu.SemaphoreType.DMA,
    ],
)


def _mlp_body(stats_ref, u_ref, W1_ref, b1_ref, g_ref, bb_ref, W2_ref, b2_ref,
              out_ref):
    stats = stats_ref[...]
    var = stats[:, NODE_DIM:2 * NODE_DIM]
    std = jnp.sqrt(jnp.maximum(var, 0.0) + 1e-5)
    aggr = jnp.concatenate(
        [stats[:, :NODE_DIM], std, stats[:, 2 * NODE_DIM:]], axis=1)
    h = jnp.dot(aggr, W1_ref[...], preferred_element_type=jnp.float32,
                precision=lax.Precision.HIGHEST) + b1_ref[...]
    alpha = 1.6732632423543772
    scale = 1.0507009873554805
    h = scale * jnp.where(h > 0, h, alpha * (jnp.exp(h) - 1.0))
    mu = jnp.mean(h, axis=-1, keepdims=True)
    v = jnp.mean((h - mu) ** 2, axis=-1, keepdims=True)
    h = (h - mu) / jnp.sqrt(v + 1e-5) * g_ref[...] + bb_ref[...]
    head = jnp.dot(h, W2_ref[...], preferred_element_type=jnp.float32,
                   precision=lax.Precision.HIGHEST) + b2_ref[...]
    out_ref[:, :G_DIM] = head
    out_ref[:, G_DIM:] = u_ref[:, G_DIM - TAIL_DIM:]


_mlp = pl.pallas_call(
    _mlp_body,
    out_shape=jax.ShapeDtypeStruct((NUM_GRAPHS, G_DIM + TAIL_DIM), jnp.float32),
)


def kernel(x, edge_index, edge_attr, u, batch, W1, b1, ln_g, ln_b, W2, b2):
    widx = 4 * jnp.arange(32, dtype=jnp.int32)[:, None] + \
        jnp.arange(8, dtype=jnp.int32)[None, :]
    gids_win = jnp.clip(widx, 0, NUM_GRAPHS).astype(batch.dtype)
    starts_win = jnp.sum(batch[None, None, :] < gids_win[:, :, None], axis=-1,
                         dtype=jnp.int32).reshape(-1)
    starts_win = jnp.concatenate(
        [starts_win, jnp.full((8,), N_NODES, jnp.int32)])
    stats = _sc_stats(x, starts_win)
    return _mlp(stats, u, W1, b1, ln_g, ln_b, W2, b2)
